# baseline (speedup 1.0000x reference)
import functools

import jax
import jax.numpy as jnp
import numpy as np
from jax import lax
from jax.experimental import pallas as pl
from jax.experimental.pallas import tpu as pltpu
from jax.experimental.pallas import tpu_sc as plsc

NUM_CORES = 2
NUM_SUBCORES = 16
NW = NUM_CORES * NUM_SUBCORES
LANES = 16

F = 26
NF = 13
V = 100000
D = 64
B = 4096
B_W = B // NW
ROWS_W = B_W * NF
CB = 16
CROWS = CB * NF
NCHUNK = B_W // CB

_OFFS13 = (np.arange(16, dtype=np.int64) * V).astype(np.int32)

_mesh = plsc.VectorSubcoreMesh(core_axis_name="c", subcore_axis_name="s")


def _make_half(f0):
    @functools.partial(
        pl.kernel,
        mesh=_mesh,
        compiler_params=pltpu.CompilerParams(use_tc_tiling_on_sc=True),
        out_type=jax.ShapeDtypeStruct((B, NF, D), jnp.float32),
        scratch_types=[
            pltpu.VMEM((B_W, 128), jnp.int32),
            pltpu.VMEM((16,), jnp.int32),
            pltpu.VMEM((ROWS_W + 16,), jnp.int32),
            pltpu.VMEM((CB, NF, D), jnp.float32),
            pltpu.VMEM((CB, NF, D), jnp.float32),
            pltpu.SemaphoreType.DMA,
            pltpu.SemaphoreType.DMA,
        ],
        name=f"sc_gather_f{f0}",
    )
    def _half(x_hbm, offs_hbm, tab_hbm, out_hbm,
              xblk_v, offs_v, xflat, buf0, buf1, gsem0, gsem1):
        wid = lax.axis_index("s") * NUM_CORES + lax.axis_index("c")
        b_base = wid * B_W

        pltpu.sync_copy(x_hbm.at[pl.ds(b_base, B_W), :], xblk_v)
        pltpu.sync_copy(offs_hbm, offs_v)
        offs = offs_v[...]

        def repack_body(r, _):
            xflat[pl.ds(r * NF, LANES)] = xblk_v[r, pl.ds(f0, LANES)] + offs
            return 0

        lax.fori_loop(0, B_W, repack_body, 0)

        def gather_start(c, buf, sem):
            def group_body(g, _):
                q0 = g * LANES
                vec = xflat[pl.ds(c * CROWS + q0, LANES)]
                for l in range(LANES):
                    r = vec[l]
                    q = q0 + l
                    bq = q // NF
                    fq = q - bq * NF
                    pltpu.make_async_copy(
                        tab_hbm.at[r], buf.at[bq, fq], sem).start()
                return 0
            lax.fori_loop(0, CROWS // LANES, group_body, 0)

        def gather_wait(buf, sem):
            pltpu.make_async_copy(
                out_hbm.at[pl.ds(0, CB), :, :], buf, sem).wait()

        def scatter(c, buf):
            pltpu.sync_copy(buf, out_hbm.at[pl.ds(b_base + c * CB, CB), :, :])

        gather_start(0, buf0, gsem0)
        gather_start(1, buf1, gsem1)

        def loop_body(i, _):
            for b, (buf, sem) in enumerate(((buf0, gsem0), (buf1, gsem1))):
                c = 2 * i + b
                gather_wait(buf, sem)
                scatter(c, buf)
                gather_start(c + 2, buf, sem)
            return 0

        lax.fori_loop(0, (NCHUNK - 2) // 2, loop_body, 0)

        for b, (buf, sem) in enumerate(((buf0, gsem0), (buf1, gsem1))):
            c = NCHUNK - 2 + b
            gather_wait(buf, sem)
            scatter(c, buf)

    return _half


_half0 = _make_half(0)
_half1 = _make_half(NF)


def kernel(X, tables):
    x = jnp.pad(jnp.asarray(X, jnp.int32), ((0, 0), (0, 128 - F)))
    offs = jnp.asarray(_OFFS13)
    tab_a = tables[:NF].reshape(NF * V, D)
    tab_b = tables[NF:].reshape(NF * V, D)
    out_a = _half0(x, offs, tab_a)
    out_b = _half1(x, offs, tab_b)
    return jnp.concatenate([out_a, out_b], axis=1)


# speedup vs baseline: 1.1154x; 1.1154x over previous
import functools

import jax
import jax.numpy as jnp
from jax import lax
from jax.experimental import pallas as pl
from jax.experimental.pallas import tpu as pltpu
from jax.experimental.pallas import tpu_sc as plsc

NUM_CORES = 2
NUM_SUBCORES = 16
NW = NUM_CORES * NUM_SUBCORES
LANES = 16

F = 26
V = 100000
D = 64
B = 4096
B_W = B // NW

_mesh = plsc.VectorSubcoreMesh(core_axis_name="c", subcore_axis_name="s")


@functools.partial(
    pl.kernel,
    mesh=_mesh,
    compiler_params=pltpu.CompilerParams(use_tc_tiling_on_sc=True),
    out_type=jax.ShapeDtypeStruct((F, D, B), jnp.float32),
    scratch_types=[
        pltpu.VMEM((B_W,), jnp.int32),
        pltpu.VMEM((B_W,), jnp.int32),
        pltpu.VMEM((D, B_W), jnp.float32),
        pltpu.VMEM((D, B_W), jnp.float32),
        pltpu.SemaphoreType.DMA,
        pltpu.SemaphoreType.DMA,
    ],
)
def _sc_gather(xt_hbm, tab_hbm, out_hbm,
               idx0, idx1, buf0, buf1, gsem0, gsem1):
    wid = lax.axis_index("s") * NUM_CORES + lax.axis_index("c")
    b_base = wid * B_W

    def gather_start(f, idx_v, buf, sem):
        pltpu.sync_copy(xt_hbm.at[f, pl.ds(b_base, B_W)], idx_v)
        foff = f * V

        def group_body(g, _):
            q0 = g * LANES
            vec = idx_v[pl.ds(q0, LANES)]
            for l in range(LANES):
                r = vec[l] + foff
                pltpu.make_async_copy(
                    tab_hbm.at[r], buf.at[:, q0 + l], sem).start()
            return 0
        lax.fori_loop(0, B_W // LANES, group_body, 0)

    def gather_wait(buf, sem):
        pltpu.make_async_copy(out_hbm.at[0, :, pl.ds(0, B_W)], buf, sem).wait()

    def scatter(f, buf):
        pltpu.sync_copy(buf, out_hbm.at[f, :, pl.ds(b_base, B_W)])

    gather_start(0, idx0, buf0, gsem0)
    gather_start(1, idx1, buf1, gsem1)

    def loop_body(i, _):
        for b, (idx_v, buf, sem) in enumerate(((idx0, buf0, gsem0),
                                               (idx1, buf1, gsem1))):
            f = 2 * i + b
            gather_wait(buf, sem)
            scatter(f, buf)
            gather_start(f + 2, idx_v, buf, sem)
        return 0

    lax.fori_loop(0, (F - 2) // 2, loop_body, 0)

    for b, (idx_v, buf, sem) in enumerate(((idx0, buf0, gsem0),
                                           (idx1, buf1, gsem1))):
        f = F - 2 + b
        gather_wait(buf, sem)
        scatter(f, buf)


def kernel(X, tables):
    xt = jnp.transpose(jnp.asarray(X, jnp.int32))
    tab = tables.reshape(F * V, D)
    out_t = _sc_gather(xt, tab)
    return jnp.transpose(out_t, (2, 0, 1))


# speedup vs baseline: 2.3829x; 2.1364x over previous
import functools

import jax
import jax.numpy as jnp
import numpy as np
from jax import lax
from jax.experimental import pallas as pl
from jax.experimental.pallas import tpu as pltpu
from jax.experimental.pallas import tpu_sc as plsc

NUM_CORES = 2
NUM_SUBCORES = 16
NW = NUM_CORES * NUM_SUBCORES
LANES = 16

F = 26
V = 100000
D = 64
B = 4096
BH = B // 2
B_W = BH // NW
ROWS_W = B_W * F
CB = 8
CROWS = CB * F
NCHUNK = B_W // CB

_OFFPAIR = np.concatenate([
    np.arange(16, dtype=np.int64) * V,
    np.arange(10, 26, dtype=np.int64) * V,
]).astype(np.int32)

_mesh = plsc.VectorSubcoreMesh(core_axis_name="c", subcore_axis_name="s")


def _make_half(half):
    @functools.partial(
        pl.kernel,
        mesh=_mesh,
        compiler_params=pltpu.CompilerParams(use_tc_tiling_on_sc=True),
        out_type=jax.ShapeDtypeStruct((BH, F, D), jnp.float32),
        scratch_types=[
            pltpu.VMEM((B_W, 128), jnp.int32),
            pltpu.VMEM((32,), jnp.int32),
            pltpu.VMEM((ROWS_W,), jnp.int32),
            pltpu.VMEM((CB, F, D), jnp.float32),
            pltpu.VMEM((CB, F, D), jnp.float32),
            pltpu.SemaphoreType.DMA,
            pltpu.SemaphoreType.DMA,
        ],
        name=f"sc_gather_h{half}",
    )
    def _half(x_hbm, offpair_hbm, tab_hbm, out_hbm,
              xblk_v, offpair_v, xflat, buf0, buf1, gsem0, gsem1):
        wid = lax.axis_index("s") * NUM_CORES + lax.axis_index("c")
        b_base = wid * B_W
        x_base = half * BH + b_base

        pltpu.sync_copy(x_hbm.at[pl.ds(x_base, B_W), :], xblk_v)
        pltpu.sync_copy(offpair_hbm, offpair_v)

        offa = offpair_v[pl.ds(0, LANES)]
        offb = offpair_v[pl.ds(LANES, LANES)]

        def repack_body(r, _):
            p = r * F
            xflat[pl.ds(p, LANES)] = xblk_v[r, pl.ds(0, LANES)] + offa
            xflat[pl.ds(p + 10, LANES)] = xblk_v[r, pl.ds(10, LANES)] + offb
            return 0

        lax.fori_loop(0, B_W, repack_body, 0)

        def gather_start(c, buf, sem):
            def group_body(g, _):
                q0 = g * LANES
                vec = xflat[pl.ds(c * CROWS + q0, LANES)]
                for l in range(LANES):
                    r = vec[l]
                    q = q0 + l
                    bq = q // F
                    fq = q - bq * F
                    pltpu.make_async_copy(
                        tab_hbm.at[r], buf.at[bq, fq], sem).start()
                return 0
            lax.fori_loop(0, CROWS // LANES, group_body, 0)

        def gather_wait(buf, sem):
            pltpu.make_async_copy(
                out_hbm.at[pl.ds(0, CB), :, :], buf, sem).wait()

        def scatter(c, buf):
            pltpu.sync_copy(buf, out_hbm.at[pl.ds(b_base + c * CB, CB), :, :])

        gather_start(0, buf0, gsem0)
        gather_start(1, buf1, gsem1)

        def loop_body(i, _):
            for b, (buf, sem) in enumerate(((buf0, gsem0), (buf1, gsem1))):
                c = 2 * i + b
                gather_wait(buf, sem)
                scatter(c, buf)
                gather_start(c + 2, buf, sem)
            return 0

        lax.fori_loop(0, (NCHUNK - 2) // 2, loop_body, 0)

        for b, (buf, sem) in enumerate(((buf0, gsem0), (buf1, gsem1))):
            c = NCHUNK - 2 + b
            gather_wait(buf, sem)
            scatter(c, buf)

    return _half


_half0 = _make_half(0)
_half1 = _make_half(1)


def kernel(X, tables):
    x = jnp.pad(jnp.asarray(X, jnp.int32), ((0, 0), (0, 128 - F)))
    tab = tables.reshape(F * V, D)
    offs = jnp.asarray(_OFFPAIR)
    out_a = _half0(x, offs, tab)
    out_b = _half1(x, offs, tab)
    return jnp.concatenate([out_a, out_b], axis=0)


# speedup vs baseline: 2.4456x; 1.0263x over previous
import functools

import jax
import jax.numpy as jnp
import numpy as np
from jax import lax
from jax.experimental import pallas as pl
from jax.experimental.pallas import tpu as pltpu
from jax.experimental.pallas import tpu_sc as plsc

NUM_CORES = 2
NUM_SUBCORES = 16
NW = NUM_CORES * NUM_SUBCORES
LANES = 16

F = 26
V = 100000
D = 64
B = 4096
B_W = B // NW
ROWS_W = B_W * F
CB = 8
CROWS = CB * F
NCHUNK = B_W // CB

_OFFPAIR = np.concatenate([
    np.arange(16, dtype=np.int64) * V,
    np.arange(10, 26, dtype=np.int64) * V,
]).astype(np.int32)

_mesh = plsc.VectorSubcoreMesh(core_axis_name="c", subcore_axis_name="s")


@functools.partial(
    pl.kernel,
    mesh=_mesh,
    compiler_params=pltpu.CompilerParams(use_tc_tiling_on_sc=True),
    out_type=jax.ShapeDtypeStruct((B, F, D), jnp.float32),
    scratch_types=[
        pltpu.VMEM((B_W, 128), jnp.int32),
        pltpu.VMEM((32,), jnp.int32),
        pltpu.VMEM((ROWS_W,), jnp.int32),
        pltpu.VMEM((CB, F, D), jnp.float32),
        pltpu.VMEM((CB, F, D), jnp.float32),
        pltpu.SemaphoreType.DMA,
        pltpu.SemaphoreType.DMA,
    ],
)
def _sc_gather(x_hbm, offpair_hbm, tab_hbm, out_hbm,
               xblk_v, offpair_v, xflat, buf0, buf1, gsem0, gsem1):
    wid = lax.axis_index("s") * NUM_CORES + lax.axis_index("c")
    b_base = wid * B_W

    pltpu.sync_copy(x_hbm.at[pl.ds(b_base, B_W), :], xblk_v)
    pltpu.sync_copy(offpair_hbm, offpair_v)

    offa = offpair_v[pl.ds(0, LANES)]
    offb = offpair_v[pl.ds(LANES, LANES)]

    def repack_body(r, _):
        p = r * F
        xflat[pl.ds(p, LANES)] = xblk_v[r, pl.ds(0, LANES)] + offa
        xflat[pl.ds(p + 10, LANES)] = xblk_v[r, pl.ds(10, LANES)] + offb
        return 0

    lax.fori_loop(0, B_W, repack_body, 0)

    def gather_start(c, buf, sem):
        def group_body(g, _):
            q0 = g * LANES
            vec = xflat[pl.ds(c * CROWS + q0, LANES)]
            for l in range(LANES):
                r = vec[l]
                q = q0 + l
                bq = q // F
                fq = q - bq * F
                pltpu.make_async_copy(
                    tab_hbm.at[r], buf.at[bq, fq], sem).start()
            return 0
        lax.fori_loop(0, CROWS // LANES, group_body, 0)

    def gather_wait(buf, sem):
        pltpu.make_async_copy(out_hbm.at[pl.ds(0, CB), :, :], buf, sem).wait()

    def scatter(c, buf):
        pltpu.sync_copy(buf, out_hbm.at[pl.ds(b_base + c * CB, CB), :, :])

    gather_start(0, buf0, gsem0)
    gather_start(1, buf1, gsem1)

    def loop_body(i, _):
        for b, (buf, sem) in enumerate(((buf0, gsem0), (buf1, gsem1))):
            c = 2 * i + b
            gather_wait(buf, sem)
            scatter(c, buf)
            gather_start(c + 2, buf, sem)
        return 0

    lax.fori_loop(0, (NCHUNK - 2) // 2, loop_body, 0)

    for b, (buf, sem) in enumerate(((buf0, gsem0), (buf1, gsem1))):
        c = NCHUNK - 2 + b
        gather_wait(buf, sem)
        scatter(c, buf)


def kernel(X, tables):
    x = jnp.pad(jnp.asarray(X, jnp.int32), ((0, 0), (0, 128 - F)))
    tab = tables.reshape(F * V, D)
    return _sc_gather(x, jnp.asarray(_OFFPAIR), tab)
